# Initial kernel scaffold; baseline (speedup 1.0000x reference)
import jax
import jax.numpy as jnp
from jax.experimental import pallas as pl


def kernel(x, edge_index, W1, b1, W2, b2):
    raise NotImplementedError("write your pallas kernel here")


# speedup vs baseline: 31.4564x; 31.4564x over previous
import functools

import jax
import jax.numpy as jnp
from jax import lax
from jax.experimental import pallas as pl
from jax.experimental.pallas import tpu as pltpu
from jax.experimental.pallas import tpu_sc as plsc

N = 10000
E = 320000
D = 128

NC = 2
NS = 16
NW = NC * NS
K = 128
NCHUNK = 80
EPT = NCHUNK * K
EP = NW * EPT
NP = 10112
RPT = NP // NS
BM = 1264


def _mesh():
    return plsc.VectorSubcoreMesh(core_axis_name="c", subcore_axis_name="s")


@functools.partial(
    pl.kernel,
    out_type=jax.ShapeDtypeStruct((NC, NP, 16), jnp.float32),
    mesh=_mesh(),
    scratch_types=[
        pltpu.VMEM((NCHUNK, K), jnp.int32),
        pltpu.VMEM((K, 16), jnp.float32),
        pltpu.VMEM((K, 16), jnp.float32),
        pltpu.VMEM_SHARED((NP, 16), jnp.float32),
    ],
)
def _sc_deg(dst_hbm, out_hbm, dst_v, ones_v, zero_v, acc):
    cid = lax.axis_index("c")
    sid = lax.axis_index("s")
    wid = sid * NC + cid
    pltpu.sync_copy(dst_hbm.at[wid], dst_v)

    def fill(i, _):
        ones_v[i] = jnp.ones((16,), jnp.float32)
        zero_v[i] = jnp.zeros((16,), jnp.float32)
        return 0

    lax.fori_loop(0, K, fill, 0)
    base = sid * RPT
    for j in range(4):
        pltpu.sync_copy(zero_v, acc.at[pl.ds(base + j * K, K)])
    pltpu.sync_copy(zero_v.at[pl.ds(0, RPT - 4 * K)],
                    acc.at[pl.ds(base + 4 * K, RPT - 4 * K)])
    plsc.subcore_barrier()

    def chunk(c, _):
        pltpu.sync_copy(ones_v, acc.at[dst_v.at[c]], add=True)
        return 0

    lax.fori_loop(0, NCHUNK, chunk, 0)
    plsc.subcore_barrier()
    pltpu.sync_copy(acc.at[pl.ds(base, RPT)],
                    out_hbm.at[cid, pl.ds(base, RPT)])


IB = 16
NB = NCHUNK // IB


@functools.partial(
    pl.kernel,
    out_type=jax.ShapeDtypeStruct((NC, NP, D), jnp.float32),
    mesh=_mesh(),
    scratch_types=[
        pltpu.VMEM((IB, K), jnp.int32),
        pltpu.VMEM((IB, K), jnp.int32),
        pltpu.VMEM((IB, K), jnp.int32),
        pltpu.VMEM((IB, K), jnp.int32),
        pltpu.VMEM((K, D), jnp.float32),
        pltpu.VMEM((K, D), jnp.float32),
        pltpu.VMEM_SHARED((NP, D), jnp.float32),
        pltpu.SemaphoreType.DMA,
        pltpu.SemaphoreType.DMA,
        pltpu.SemaphoreType.DMA,
    ],
)
def _sc_msgpass(y_hbm, src_hbm, dst_hbm, out_hbm,
                srcb0, dstb0, srcb1, dstb1, buf0, buf1, acc,
                sem0, sem1, semi):
    cid = lax.axis_index("c")
    sid = lax.axis_index("s")
    wid = sid * NC + cid

    def zrow(i, _):
        for c8 in range(D // 16):
            buf0[i, pl.ds(16 * c8, 16)] = jnp.zeros((16,), jnp.float32)
        return 0

    lax.fori_loop(0, K, zrow, 0)
    base = sid * RPT
    for j in range(4):
        pltpu.sync_copy(buf0, acc.at[pl.ds(base + j * K, K)])
    pltpu.sync_copy(buf0.at[pl.ds(0, RPT - 4 * K)],
                    acc.at[pl.ds(base + 4 * K, RPT - 4 * K)])
    pltpu.sync_copy(src_hbm.at[wid, pl.ds(0, IB)], srcb0)
    pltpu.sync_copy(dst_hbm.at[wid, pl.ds(0, IB)], dstb0)
    plsc.subcore_barrier()

    for s in range(NB):
        srcb, dstb = (srcb0, dstb0) if s % 2 == 0 else (srcb1, dstb1)
        nsrcb, ndstb = (srcb1, dstb1) if s % 2 == 0 else (srcb0, dstb0)
        if s + 1 < NB:
            pltpu.async_copy(src_hbm.at[wid, pl.ds((s + 1) * IB, IB)],
                             nsrcb, semi)
            pltpu.async_copy(dst_hbm.at[wid, pl.ds((s + 1) * IB, IB)],
                             ndstb, semi)
        pltpu.async_copy(y_hbm.at[srcb.at[0]], buf0, sem0)
        pltpu.async_copy(y_hbm.at[srcb.at[1]], buf1, sem1)

        def pair(p, _, srcb=srcb, dstb=dstb):
            c = p * 2
            for b in range(2):
                buf = buf0 if b == 0 else buf1
                sem = sem0 if b == 0 else sem1
                cc = c + b
                pltpu.make_async_copy(y_hbm.at[srcb.at[cc]], buf, sem).wait()
                pltpu.sync_copy(buf, acc.at[dstb.at[cc]], add=True)

                @pl.when(cc + 2 < IB)
                def _():
                    pltpu.async_copy(y_hbm.at[srcb.at[cc + 2]], buf, sem)

            return 0

        lax.fori_loop(0, IB // 2, pair, 0)
        if s + 1 < NB:
            pltpu.make_async_copy(src_hbm.at[wid, pl.ds(0, IB)],
                                  nsrcb, semi).wait()
            pltpu.make_async_copy(dst_hbm.at[wid, pl.ds(0, IB)],
                                  ndstb, semi).wait()
    plsc.subcore_barrier()
    pltpu.sync_copy(acc.at[pl.ds(base, RPT)],
                    out_hbm.at[cid, pl.ds(base, RPT)])


def _prep_body(x_ref, w_ref, degp_ref, y_ref, dinv_ref):
    counts = jnp.sum(degp_ref[0], axis=1) + jnp.sum(degp_ref[1], axis=1)
    deg = counts * (1.0 / 16.0) + 1.0
    dinv = lax.rsqrt(deg)[:, None]
    z = jnp.dot(x_ref[...], w_ref[...], preferred_element_type=jnp.float32)
    y_ref[...] = z * dinv
    dinv_ref[...] = jnp.broadcast_to(dinv, z.shape)


def _tc_prep(x_p, W1, degp):
    return pl.pallas_call(
        _prep_body,
        grid=(NP // BM,),
        in_specs=[
            pl.BlockSpec((BM, D), lambda i: (i, 0)),
            pl.BlockSpec((D, D), lambda i: (0, 0)),
            pl.BlockSpec((NC, BM, 16), lambda i: (0, i, 0)),
        ],
        out_specs=[
            pl.BlockSpec((BM, D), lambda i: (i, 0)),
            pl.BlockSpec((BM, D), lambda i: (i, 0)),
        ],
        out_shape=[
            jax.ShapeDtypeStruct((NP, D), jnp.float32),
            jax.ShapeDtypeStruct((NP, D), jnp.float32),
        ],
    )(x_p, W1, degp)


def _mid_body(p_ref, y_ref, dinv_ref, b_ref, w_ref, y2_ref):
    s = (p_ref[0] + p_ref[1] + y_ref[...]) * dinv_ref[...] + b_ref[...]
    h = jnp.maximum(s, 0.0)
    i = pl.program_id(0)
    rows = lax.broadcasted_iota(jnp.int32, h.shape, 0) + i * BM
    h = jnp.where(rows < N, h, 0.0)
    y2_ref[...] = jnp.dot(h, w_ref[...],
                          preferred_element_type=jnp.float32) * dinv_ref[...]


def _tc_mid(p1, y1, dinv_b, b1r, W2):
    return pl.pallas_call(
        _mid_body,
        grid=(NP // BM,),
        in_specs=[
            pl.BlockSpec((NC, BM, D), lambda i: (0, i, 0)),
            pl.BlockSpec((BM, D), lambda i: (i, 0)),
            pl.BlockSpec((BM, D), lambda i: (i, 0)),
            pl.BlockSpec((1, D), lambda i: (0, 0)),
            pl.BlockSpec((D, D), lambda i: (0, 0)),
        ],
        out_specs=pl.BlockSpec((BM, D), lambda i: (i, 0)),
        out_shape=jax.ShapeDtypeStruct((NP, D), jnp.float32),
    )(p1, y1, dinv_b, b1r, W2)


_BMF = 1000


def _final_body(p_ref, y_ref, dinv_ref, b_ref, out_ref):
    out_ref[...] = ((p_ref[0] + p_ref[1] + y_ref[...]) * dinv_ref[...]
                    + b_ref[...])


def _tc_final(p2, y2, dinv_b, b2r):
    return pl.pallas_call(
        _final_body,
        grid=(N // _BMF,),
        in_specs=[
            pl.BlockSpec((NC, _BMF, D), lambda i: (0, i, 0)),
            pl.BlockSpec((_BMF, D), lambda i: (i, 0)),
            pl.BlockSpec((_BMF, D), lambda i: (i, 0)),
            pl.BlockSpec((1, D), lambda i: (0, 0)),
        ],
        out_specs=pl.BlockSpec((_BMF, D), lambda i: (i, 0)),
        out_shape=jax.ShapeDtypeStruct((N, D), jnp.float32),
    )(p2, y2, dinv_b, b2r)


def kernel(x, edge_index, W1, b1, W2, b2):
    ei = edge_index.astype(jnp.int32)
    npad = EP - E
    pad = N + (lax.iota(jnp.int32, npad) % (NP - N))
    src_p = jnp.concatenate([ei[0], pad]).reshape(NW, NCHUNK, K)
    dst_p = jnp.concatenate([ei[1], pad]).reshape(NW, NCHUNK, K)
    x_p = jnp.pad(x, ((0, NP - N), (0, 0)))

    degp = _sc_deg(dst_p)
    y1, dinv_b = _tc_prep(x_p, W1, degp)
    p1 = _sc_msgpass(y1, src_p, dst_p)
    y2 = _tc_mid(p1, y1, dinv_b, b1.reshape(1, D), W2)
    p2 = _sc_msgpass(y2, src_p, dst_p)
    return _tc_final(p2, y2, dinv_b, b2.reshape(1, D))
